# baseline (speedup 1.0000x reference)
import functools

import jax
import jax.numpy as jnp
from jax import lax
from jax.experimental import pallas as pl
from jax.experimental.pallas import tpu as pltpu
from jax.experimental.pallas import tpu_sc as plsc

DIM = 256
N_TOK = 32768
N_EMB = 8192
TM = 256
LANES = 128
NCH = N_EMB // LANES
R = N_TOK // TM


def _dist_body(x_ref, e_ref, dist_ref, ind_ref, lsum_ref, e2_ref):
    i = pl.program_id(0)

    @pl.when(i == 0)
    def _():
        e = e_ref[...]
        e2_ref[...] = jnp.sum(e * e, axis=0, keepdims=True)
        lsum_ref[0, 0] = 0.0

    x = x_ref[...]
    x2 = jnp.sum(x * x, axis=1, keepdims=True)
    xe2 = jnp.dot(x * -2.0, e_ref[...], preferred_element_type=jnp.float32)
    e2 = e2_ref[...]

    RS = TM // 2
    for h in range(2):
        r0, r1 = h * RS, (h + 1) * RS
        x2h = x2[r0:r1]
        bv = None
        bk = None
        for k in range(NCH):
            lo, hi = k * LANES, (k + 1) * LANES
            d = (x2h + xe2[r0:r1, lo:hi]) + e2[:, lo:hi]
            dist_ref[r0:r1, lo:hi] = d
            if k == 0:
                bv = d
                bk = jnp.zeros((RS, LANES), jnp.int32)
            else:
                upd = d < bv
                bv = jnp.where(upd, d, bv)
                bk = jnp.where(upd, jnp.int32(k), bk)

        rowmin = jnp.min(bv, axis=1, keepdims=True)
        idx = bk * LANES + jax.lax.broadcasted_iota(
            jnp.int32, (RS, LANES), 1)
        cand = jnp.where(bv == rowmin, idx, jnp.int32(N_EMB))
        ind_ref[r0:r1] = jnp.min(cand, axis=1)
        lsum_ref[0, 0] += jnp.sum(rowmin)


def _dist_argmin(x, embed):
    return pl.pallas_call(
        _dist_body,
        grid=(R,),
        in_specs=[
            pl.BlockSpec((TM, DIM), lambda i: (i, 0)),
            pl.BlockSpec((DIM, N_EMB), lambda i: (0, 0)),
        ],
        out_specs=[
            pl.BlockSpec((TM, N_EMB), lambda i: (i, 0)),
            pl.BlockSpec((TM,), lambda i: (i,)),
            pl.BlockSpec(memory_space=pltpu.SMEM),
        ],
        out_shape=[
            jax.ShapeDtypeStruct((N_TOK, N_EMB), jnp.float32),
            jax.ShapeDtypeStruct((N_TOK,), jnp.int32),
            jax.ShapeDtypeStruct((1, 1), jnp.float32),
        ],
        scratch_shapes=[
            pltpu.VMEM((1, N_EMB), jnp.float32),
        ],
        compiler_params=pltpu.CompilerParams(
            dimension_semantics=("arbitrary",)),
    )(x, embed)


_SC_INFO = plsc.get_sparse_core_info()
_NC = _SC_INFO.num_cores
_NW = _NC * _SC_INFO.num_subcores
B_PER_W = N_TOK // _NW
CHUNK = 128
NCHUNKS = B_PER_W // CHUNK


@functools.partial(
    pl.kernel,
    mesh=plsc.VectorSubcoreMesh(core_axis_name="c", subcore_axis_name="s"),
    out_type=jax.ShapeDtypeStruct((N_TOK, DIM), jnp.float32),
    scratch_types=[
        pltpu.VMEM((B_PER_W,), jnp.int32),
        pltpu.VMEM((CHUNK, DIM), jnp.float32),
        pltpu.VMEM((CHUNK, DIM), jnp.float32),
        pltpu.VMEM((CHUNK, DIM), jnp.float32),
        pltpu.SemaphoreType.DMA,
        pltpu.SemaphoreType.DMA,
        pltpu.SemaphoreType.DMA,
        pltpu.SemaphoreType.DMA,
        pltpu.SemaphoreType.DMA,
        pltpu.SemaphoreType.DMA,
    ],
)
def _gather_rows(table_hbm, idx_hbm, out_hbm, idx_v, rows_a, rows_b, rows_c,
                 sem_a, sem_b, sem_c, osem_a, osem_b, osem_c):
    wid = lax.axis_index("s") * _NC + lax.axis_index("c")
    base = wid * B_PER_W
    pltpu.sync_copy(idx_hbm.at[pl.ds(base, B_PER_W)], idx_v)
    bufs = (rows_a, rows_b, rows_c)
    sems = (sem_a, sem_b, sem_c)
    osems = (osem_a, osem_b, osem_c)
    ND = len(bufs)
    gh = [None] * NCHUNKS
    sh = [None] * NCHUNKS
    for c in range(NCHUNKS + ND - 1):
        if c < NCHUNKS:
            if c >= ND:
                sh[c - ND].wait()
            gh[c] = pltpu.async_copy(
                table_hbm.at[idx_v.at[pl.ds(c * CHUNK, CHUNK)]],
                bufs[c % ND], sems[c % ND])
        w = c - ND + 1
        if 0 <= w < NCHUNKS:
            gh[w].wait()
            sh[w] = pltpu.async_copy(
                bufs[w % ND],
                out_hbm.at[pl.ds(base + w * CHUNK, CHUNK)],
                osems[w % ND])
    for w in range(max(0, NCHUNKS - ND), NCHUNKS):
        sh[w].wait()


def kernel(input, embed):
    x = input.reshape(-1, DIM)
    dist, ind, lsum = _dist_argmin(x, embed)
    table = embed.T
    q = _gather_rows(table, ind)
    loss = lsum[0, 0] / jnp.float32(N_TOK * DIM)
    return (q.reshape(input.shape), ind.reshape(input.shape[:-1]), loss, dist)


# speedup vs baseline: 1.0555x; 1.0555x over previous
import functools

import jax
import jax.numpy as jnp
from jax import lax
from jax.experimental import pallas as pl
from jax.experimental.pallas import tpu as pltpu
from jax.experimental.pallas import tpu_sc as plsc

DIM = 256
N_TOK = 32768
N_EMB = 8192
TM = 512
LANES = 128
NCH = N_EMB // LANES
R = N_TOK // TM


def _dist_body(x_ref, e_ref, dist_ref, ind_ref, lsum_ref, e2_ref):
    i = pl.program_id(0)

    @pl.when(i == 0)
    def _():
        e = e_ref[...]
        e2_ref[...] = jnp.sum(e * e, axis=0, keepdims=True)
        lsum_ref[0, 0] = 0.0

    x = x_ref[...]
    x2 = jnp.sum(x * x, axis=1, keepdims=True)
    xe2 = jnp.dot(x * -2.0, e_ref[...], preferred_element_type=jnp.float32)
    e2 = e2_ref[...]

    RS = TM // 4
    for h in range(4):
        r0, r1 = h * RS, (h + 1) * RS
        x2h = x2[r0:r1]
        bv = None
        bk = None
        for k in range(NCH):
            lo, hi = k * LANES, (k + 1) * LANES
            d = (x2h + xe2[r0:r1, lo:hi]) + e2[:, lo:hi]
            dist_ref[r0:r1, lo:hi] = d
            if k == 0:
                bv = d
                bk = jnp.zeros((RS, LANES), jnp.int32)
            else:
                upd = d < bv
                bv = jnp.where(upd, d, bv)
                bk = jnp.where(upd, jnp.int32(k), bk)

        rowmin = jnp.min(bv, axis=1, keepdims=True)
        idx = bk * LANES + jax.lax.broadcasted_iota(
            jnp.int32, (RS, LANES), 1)
        cand = jnp.where(bv == rowmin, idx, jnp.int32(N_EMB))
        ind_ref[r0:r1] = jnp.min(cand, axis=1)
        lsum_ref[0, 0] += jnp.sum(rowmin)


def _dist_argmin(x, embed):
    return pl.pallas_call(
        _dist_body,
        grid=(R,),
        in_specs=[
            pl.BlockSpec((TM, DIM), lambda i: (i, 0)),
            pl.BlockSpec((DIM, N_EMB), lambda i: (0, 0)),
        ],
        out_specs=[
            pl.BlockSpec((TM, N_EMB), lambda i: (i, 0)),
            pl.BlockSpec((TM,), lambda i: (i,)),
            pl.BlockSpec(memory_space=pltpu.SMEM),
        ],
        out_shape=[
            jax.ShapeDtypeStruct((N_TOK, N_EMB), jnp.float32),
            jax.ShapeDtypeStruct((N_TOK,), jnp.int32),
            jax.ShapeDtypeStruct((1, 1), jnp.float32),
        ],
        scratch_shapes=[
            pltpu.VMEM((1, N_EMB), jnp.float32),
        ],
        compiler_params=pltpu.CompilerParams(
            dimension_semantics=("arbitrary",)),
    )(x, embed)


_SC_INFO = plsc.get_sparse_core_info()
_NC = _SC_INFO.num_cores
_NW = _NC * _SC_INFO.num_subcores
B_PER_W = N_TOK // _NW
CHUNK = 128
NCHUNKS = B_PER_W // CHUNK


@functools.partial(
    pl.kernel,
    mesh=plsc.VectorSubcoreMesh(core_axis_name="c", subcore_axis_name="s"),
    out_type=jax.ShapeDtypeStruct((N_TOK, DIM), jnp.float32),
    scratch_types=[
        pltpu.VMEM((B_PER_W,), jnp.int32),
        pltpu.VMEM((CHUNK, DIM), jnp.float32),
        pltpu.VMEM((CHUNK, DIM), jnp.float32),
        pltpu.VMEM((CHUNK, DIM), jnp.float32),
        pltpu.SemaphoreType.DMA,
        pltpu.SemaphoreType.DMA,
        pltpu.SemaphoreType.DMA,
        pltpu.SemaphoreType.DMA,
        pltpu.SemaphoreType.DMA,
        pltpu.SemaphoreType.DMA,
    ],
)
def _gather_rows(table_hbm, idx_hbm, out_hbm, idx_v, rows_a, rows_b, rows_c,
                 sem_a, sem_b, sem_c, osem_a, osem_b, osem_c):
    wid = lax.axis_index("s") * _NC + lax.axis_index("c")
    base = wid * B_PER_W
    pltpu.sync_copy(idx_hbm.at[pl.ds(base, B_PER_W)], idx_v)
    bufs = (rows_a, rows_b, rows_c)
    sems = (sem_a, sem_b, sem_c)
    osems = (osem_a, osem_b, osem_c)
    ND = len(bufs)
    gh = [None] * NCHUNKS
    sh = [None] * NCHUNKS
    for c in range(NCHUNKS + ND - 1):
        if c < NCHUNKS:
            if c >= ND:
                sh[c - ND].wait()
            gh[c] = pltpu.async_copy(
                table_hbm.at[idx_v.at[pl.ds(c * CHUNK, CHUNK)]],
                bufs[c % ND], sems[c % ND])
        w = c - ND + 1
        if 0 <= w < NCHUNKS:
            gh[w].wait()
            sh[w] = pltpu.async_copy(
                bufs[w % ND],
                out_hbm.at[pl.ds(base + w * CHUNK, CHUNK)],
                osems[w % ND])
    for w in range(max(0, NCHUNKS - ND), NCHUNKS):
        sh[w].wait()


def kernel(input, embed):
    x = input.reshape(-1, DIM)
    dist, ind, lsum = _dist_argmin(x, embed)
    table = embed.T
    q = _gather_rows(table, ind)
    loss = lsum[0, 0] / jnp.float32(N_TOK * DIM)
    return (q.reshape(input.shape), ind.reshape(input.shape[:-1]), loss, dist)
